# baseline (speedup 1.0000x reference)
import math

import jax
import jax.numpy as jnp
from jax import lax
from jax.experimental import pallas as pl
from jax.experimental.pallas import tpu as pltpu

N = 8
P = 512
NY = 64
NX = 64
JH = 16
INV_STEP = 1.0 / 64.0
ROWS = NY * NX * (P // 128)
HROWS = ROWS // 4
NSTEP = N * 4
NBUF = 8


def _phi_body(var_ref, bd_ref, out_ref, buf, sem):
    m = pl.program_id(0)
    p = m % NBUF
    h = (m % 4).astype(jnp.float32)

    var = var_ref[0, 0]
    inv2s2 = 1.0 / (2.0 * var * var)
    norm = 1.0 / (2.0 * math.pi * var * var)

    b = bd_ref[0, 0:1, :]
    q = bd_ref[0, 1:2, :] - b

    xv = lax.broadcasted_iota(jnp.int32, (NX, P), 0).astype(jnp.float32) * INV_STEP
    yv = lax.broadcasted_iota(jnp.int32, (JH, P), 0).astype(jnp.float32) * INV_STEP \
        + h * (JH * INV_STEP)
    gx = jnp.exp(-jnp.square(xv - b) * inv2s2) * norm
    gy = jnp.exp(-jnp.square(yv - q) * inv2s2)

    qx = gx.reshape(NX * 4, 128)
    qy = gy.reshape(JH * 4, 128)
    vy = jnp.broadcast_to(
        qy.reshape(JH, 1, 4, 128), (JH, 2, 4, 128)
    ).reshape(JH, 8, 128)

    prod = vy.reshape(JH, 1, 8, 128) * qx.reshape(1, NX // 2, 8, 128)

    @pl.when(m >= NBUF)
    def _reclaim():
        pltpu.make_async_copy(buf.at[p], out_ref.at[m - NBUF], sem.at[p]).wait()

    buf[p] = prod.reshape(HROWS, 128)
    pltpu.make_async_copy(buf.at[p], out_ref.at[m], sem.at[p]).start()

    @pl.when(m == NSTEP - 1)
    def _drain():
        for k in range(NBUF - 1, -1, -1):
            pk = (m - k) % NBUF
            pltpu.make_async_copy(buf.at[pk], out_ref.at[m - k], sem.at[pk]).wait()


def kernel(diagrams, variance):
    bd = diagrams.transpose(0, 2, 1)
    var = jnp.reshape(variance, (1, 1)).astype(jnp.float32)

    out = pl.pallas_call(
        _phi_body,
        grid=(NSTEP,),
        in_specs=[
            pl.BlockSpec((1, 1), lambda m: (0, 0)),
            pl.BlockSpec((1, 2, P), lambda m: (m // 4, 0, 0)),
        ],
        out_specs=pl.BlockSpec(memory_space=pl.ANY),
        out_shape=jax.ShapeDtypeStruct((NSTEP, HROWS, 128), jnp.float32),
        scratch_shapes=[
            pltpu.VMEM((NBUF, HROWS, 128), jnp.float32),
            pltpu.SemaphoreType.DMA((NBUF,)),
        ],
    )(var, bd)

    return out.reshape(N, NY, NX, 1, P).transpose(0, 4, 1, 2, 3)


# speedup vs baseline: 1.1064x; 1.1064x over previous
import math

import jax
import jax.numpy as jnp
from jax import lax
from jax.experimental import pallas as pl
from jax.experimental.pallas import tpu as pltpu

N = 8
P = 512
NY = 64
NX = 64
JH = 32
INV_STEP = 1.0 / 64.0
ROWS = NY * NX * (P // 128)
HROWS = ROWS // 2
NSTEP = N * 2
NBUF = 4


def _phi_body(var_ref, bd_ref, out_ref, buf, sem):
    m = pl.program_id(0)
    p = m % NBUF
    h = (m % 2).astype(jnp.float32)

    var = var_ref[0, 0]
    inv2s2 = 1.0 / (2.0 * var * var)
    norm = 1.0 / (2.0 * math.pi * var * var)

    b = bd_ref[0, 0:1, :]
    q = bd_ref[0, 1:2, :] - b

    xv = lax.broadcasted_iota(jnp.int32, (NX, P), 0).astype(jnp.float32) * INV_STEP
    yv = lax.broadcasted_iota(jnp.int32, (JH, P), 0).astype(jnp.float32) * INV_STEP \
        + h * (JH * INV_STEP)
    gx = jnp.exp(-jnp.square(xv - b) * inv2s2) * norm
    gy = jnp.exp(-jnp.square(yv - q) * inv2s2)

    qx = gx.reshape(NX * 4, 128)
    qy = gy.reshape(JH * 4, 128)
    vy = jnp.broadcast_to(
        qy.reshape(JH, 1, 4, 128), (JH, 2, 4, 128)
    ).reshape(JH, 8, 128)

    prod = vy.reshape(JH, 1, 8, 128) * qx.reshape(1, NX // 2, 8, 128)

    @pl.when(m >= NBUF)
    def _reclaim():
        pltpu.make_async_copy(buf.at[p], out_ref.at[m - NBUF], sem.at[p]).wait()

    buf[p] = prod.reshape(HROWS, 128)
    pltpu.make_async_copy(buf.at[p], out_ref.at[m], sem.at[p]).start()

    @pl.when(m == NSTEP - 1)
    def _drain():
        for k in range(NBUF - 1, -1, -1):
            pk = (m - k) % NBUF
            pltpu.make_async_copy(buf.at[pk], out_ref.at[m - k], sem.at[pk]).wait()


def kernel(diagrams, variance):
    bd = diagrams.transpose(0, 2, 1)
    var = jnp.reshape(variance, (1, 1)).astype(jnp.float32)

    out = pl.pallas_call(
        _phi_body,
        grid=(NSTEP,),
        in_specs=[
            pl.BlockSpec((1, 1), lambda m: (0, 0)),
            pl.BlockSpec((1, 2, P), lambda m: (m // 2, 0, 0)),
        ],
        out_specs=pl.BlockSpec(memory_space=pl.ANY),
        out_shape=jax.ShapeDtypeStruct((NSTEP, HROWS, 128), jnp.float32),
        scratch_shapes=[
            pltpu.VMEM((NBUF, HROWS, 128), jnp.float32),
            pltpu.SemaphoreType.DMA((NBUF,)),
        ],
    )(var, bd)

    return out.reshape(N, NY, NX, 1, P).transpose(0, 4, 1, 2, 3)
